# baseline (speedup 1.0000x reference)
import functools

import jax
import jax.numpy as jnp
import numpy as np
from jax.experimental import pallas as pl
from jax.experimental.pallas import tpu as pltpu


def _fused_kernel(e_blk, eall_ref, x_ref, wa_ref, wb_ref, pmat_ref,
                  ssel_ref, bp_ref, lwsel_ref, lb_ref, out_ref,
                  u_ref, v_ref, *, n_rows, blk_r):
    i = pl.program_id(0)

    @pl.when(i == 0)
    def _prep():
        nb = x_ref.shape[0]
        wcols = wa_ref.shape[1]
        for b in range(nb):
            xb = x_ref[b]
            u_ref[0:n_rows, wcols * b:wcols * (b + 1)] = jnp.dot(
                xb, wa_ref[...], preferred_element_type=jnp.float32)
            v_ref[:, wcols * b:wcols * (b + 1)] = jnp.dot(
                xb, wb_ref[...], preferred_element_type=jnp.float32)
        v_ref[:, nb * wcols:nb * wcols + 1] = jnp.ones(
            (n_rows, 1), jnp.float32)

    eb = e_blk[...]
    a = jax.lax.dot_general(eb, eall_ref[...],
                            (((1,), (1,)), ((), ())),
                            preferred_element_type=jnp.float32)
    p = jnp.exp(jnp.minimum(jnp.maximum(a, 0.0), 85.0))
    pv = jnp.dot(p, v_ref[...], preferred_element_type=jnp.float32)
    ncols = v_ref.shape[1] - 1
    inv = 1.0 / pv[:, ncols:ncols + 1]

    t = u_ref[pl.ds(i * blk_r, blk_r), :] + pv[:, 0:ncols] * inv
    emul = jnp.dot(eb, pmat_ref[...], preferred_element_type=jnp.float32)
    gu = (jnp.dot(t * emul, ssel_ref[...], preferred_element_type=jnp.float32)
          + jnp.dot(eb, bp_ref[...], preferred_element_type=jnp.float32))
    r = jax.nn.sigmoid(gu[:, 0:8])
    h = jnp.tanh(gu[:, 8:16])
    y = jnp.maximum((1.0 - r) * h, 0.0)
    yo = (jnp.dot(y, lwsel_ref[...], preferred_element_type=jnp.float32)
          + lb_ref[0:1, 0:1])
    out_ref[...] = yo.T


def kernel(x, e, gate_weights_pool, gate_bias_pool, update_weights_pool,
           update_bias_pool, linear_w, linear_b):
    B, N, C = x.shape
    D = e.shape[1]
    R = 512
    ng = pl.cdiv(N, R)

    gw = gate_weights_pool[:, :, :C, :]
    uw = update_weights_pool[:, :, :C, :]
    wa1 = jnp.concatenate([
        jnp.transpose(gw[:, 0], (1, 0, 2)).reshape(C, 4 * D),
        jnp.transpose(uw[:, 0], (1, 0, 2)).reshape(C, 2 * D),
    ], axis=1)
    wb1 = jnp.concatenate([
        jnp.transpose(gw[:, 1], (1, 0, 2)).reshape(C, 4 * D),
        jnp.transpose(uw[:, 1], (1, 0, 2)).reshape(C, 2 * D),
    ], axis=1)

    pm1 = np.zeros((D, 24), np.float32)
    for d in range(D):
        pm1[d, 4 * d:4 * d + 4] = 1.0
        pm1[d, 16 + 2 * d:16 + 2 * d + 2] = 1.0
    pmat = jnp.asarray(np.tile(pm1, (1, B)))

    ss1 = np.zeros((24, 16), np.float32)
    for d in range(D):
        for j in range(2):
            ss1[4 * d + 2 + j, j] = 1.0
            ss1[16 + 2 * d + j, 8 + j] = 1.0
    ssel_np = np.zeros((B * 24, 16), np.float32)
    for b in range(B):
        ssel_np[b * 24:(b + 1) * 24, 2 * b:2 * b + 2] = ss1[:, 0:2]
        ssel_np[b * 24:(b + 1) * 24, 8 + 2 * b:8 + 2 * b + 2] = ss1[:, 8:10]
    ssel = jnp.asarray(ssel_np)

    bp = jnp.concatenate([gate_bias_pool[:, 2:4]] * B
                         + [update_bias_pool] * B, axis=1)

    lwsel = jnp.kron(jnp.eye(B, dtype=jnp.float32), linear_w.T)
    lb2 = linear_b.reshape(1, 1)

    y2 = pl.pallas_call(
        functools.partial(_fused_kernel, n_rows=N, blk_r=R),
        grid=(ng,),
        in_specs=[
            pl.BlockSpec((R, D), lambda i: (i, 0)),
            pl.BlockSpec((N, D), lambda i: (0, 0)),
            pl.BlockSpec((B, N, C), lambda i: (0, 0, 0)),
            pl.BlockSpec((C, 24), lambda i: (0, 0)),
            pl.BlockSpec((C, 24), lambda i: (0, 0)),
            pl.BlockSpec((D, 24 * B), lambda i: (0, 0)),
            pl.BlockSpec((24 * B, 4 * B), lambda i: (0, 0)),
            pl.BlockSpec((D, 4 * B), lambda i: (0, 0)),
            pl.BlockSpec((2 * B, B), lambda i: (0, 0)),
            pl.BlockSpec((1, 1), lambda i: (0, 0)),
        ],
        out_specs=pl.BlockSpec((B, R), lambda i: (0, i)),
        out_shape=jax.ShapeDtypeStruct((B, N), jnp.float32),
        scratch_shapes=[
            pltpu.VMEM((ng * R, 24 * B), jnp.float32),
            pltpu.VMEM((N, 24 * B + 1), jnp.float32),
        ],
        compiler_params=pltpu.CompilerParams(
            dimension_semantics=("arbitrary",),
        ),
    )(e, e, x, wa1, wb1, pmat, ssel, bp, lwsel, lb2)

    return y2[:, :, None]


# speedup vs baseline: 1.0003x; 1.0003x over previous
import functools

import jax
import jax.numpy as jnp
import numpy as np
from jax.experimental import pallas as pl
from jax.experimental.pallas import tpu as pltpu


def _fused_kernel(e_blk, eall_ref, x_ref, wax_ref, wbx_ref, pmat_ref,
                  ssel_ref, bp_ref, lwsel_ref, lb_ref, out_ref,
                  u_ref, v_ref, *, n_rows, blk_r, ncols):
    i = pl.program_id(0)

    @pl.when(i == 0)
    def _prep():
        nb = x_ref.shape[0]
        u_acc = jnp.dot(x_ref[0], wax_ref[0],
                        preferred_element_type=jnp.float32)
        v_acc = jnp.dot(x_ref[0], wbx_ref[0],
                        preferred_element_type=jnp.float32)
        for b in range(1, nb):
            u_acc = u_acc + jnp.dot(x_ref[b], wax_ref[b],
                                    preferred_element_type=jnp.float32)
            v_acc = v_acc + jnp.dot(x_ref[b], wbx_ref[b],
                                    preferred_element_type=jnp.float32)
        u_ref[0:n_rows, :] = u_acc
        v_ref[:, 0:ncols] = v_acc
        v_ref[:, ncols:ncols + 1] = jnp.ones((n_rows, 1), jnp.float32)

    eb = e_blk[...]
    a = jax.lax.dot_general(eb, eall_ref[...],
                            (((1,), (1,)), ((), ())),
                            preferred_element_type=jnp.float32)
    p = jnp.exp(jnp.minimum(jnp.maximum(a, 0.0), 85.0))
    pv = jnp.dot(p, v_ref[...], preferred_element_type=jnp.float32)
    inv = 1.0 / pv[:, ncols:ncols + 1]

    t = u_ref[pl.ds(i * blk_r, blk_r), :] + pv[:, 0:ncols] * inv
    emul = jnp.dot(eb, pmat_ref[...], preferred_element_type=jnp.float32)
    gu = (jnp.dot(t * emul, ssel_ref[...], preferred_element_type=jnp.float32)
          + jnp.dot(eb, bp_ref[...], preferred_element_type=jnp.float32))
    r = jax.nn.sigmoid(gu[:, 0:8])
    h = jnp.tanh(gu[:, 8:16])
    y = jnp.maximum((1.0 - r) * h, 0.0)
    yo = (jnp.dot(y, lwsel_ref[...], preferred_element_type=jnp.float32)
          + lb_ref[0:1, 0:1])
    out_ref[...] = yo.T


def kernel(x, e, gate_weights_pool, gate_bias_pool, update_weights_pool,
           update_bias_pool, linear_w, linear_b):
    B, N, C = x.shape
    D = e.shape[1]
    R = 512
    ng = pl.cdiv(N, R)

    gw = gate_weights_pool[:, :, :C, :]
    uw = update_weights_pool[:, :, :C, :]
    wa1 = jnp.concatenate([
        jnp.transpose(gw[:, 0], (1, 0, 2)).reshape(C, 4 * D),
        jnp.transpose(uw[:, 0], (1, 0, 2)).reshape(C, 2 * D),
    ], axis=1)
    wb1 = jnp.concatenate([
        jnp.transpose(gw[:, 1], (1, 0, 2)).reshape(C, 4 * D),
        jnp.transpose(uw[:, 1], (1, 0, 2)).reshape(C, 2 * D),
    ], axis=1)
    eyeb = jnp.eye(B, dtype=jnp.float32)
    wax = jnp.kron(eyeb, wa1).reshape(B, C, 24 * B)
    wbx = jnp.kron(eyeb, wb1).reshape(B, C, 24 * B)

    pm1 = np.zeros((D, 24), np.float32)
    for d in range(D):
        pm1[d, 4 * d:4 * d + 4] = 1.0
        pm1[d, 16 + 2 * d:16 + 2 * d + 2] = 1.0
    pmat = jnp.asarray(np.tile(pm1, (1, B)))

    ss1 = np.zeros((24, 16), np.float32)
    for d in range(D):
        for j in range(2):
            ss1[4 * d + 2 + j, j] = 1.0
            ss1[16 + 2 * d + j, 8 + j] = 1.0
    ssel_np = np.zeros((B * 24, 16), np.float32)
    for b in range(B):
        ssel_np[b * 24:(b + 1) * 24, 2 * b:2 * b + 2] = ss1[:, 0:2]
        ssel_np[b * 24:(b + 1) * 24, 8 + 2 * b:8 + 2 * b + 2] = ss1[:, 8:10]
    ssel = jnp.asarray(ssel_np)

    bp = jnp.concatenate([gate_bias_pool[:, 2:4]] * B
                         + [update_bias_pool] * B, axis=1)

    lwsel = jnp.kron(eyeb, linear_w.T)
    lb2 = linear_b.reshape(1, 1)

    y2 = pl.pallas_call(
        functools.partial(_fused_kernel, n_rows=N, blk_r=R, ncols=24 * B),
        grid=(ng,),
        in_specs=[
            pl.BlockSpec((R, D), lambda i: (i, 0)),
            pl.BlockSpec((N, D), lambda i: (0, 0)),
            pl.BlockSpec((B, N, C), lambda i: (0, 0, 0)),
            pl.BlockSpec((B, C, 24 * B), lambda i: (0, 0, 0)),
            pl.BlockSpec((B, C, 24 * B), lambda i: (0, 0, 0)),
            pl.BlockSpec((D, 24 * B), lambda i: (0, 0)),
            pl.BlockSpec((24 * B, 4 * B), lambda i: (0, 0)),
            pl.BlockSpec((D, 4 * B), lambda i: (0, 0)),
            pl.BlockSpec((2 * B, B), lambda i: (0, 0)),
            pl.BlockSpec((1, 1), lambda i: (0, 0)),
        ],
        out_specs=pl.BlockSpec((B, R), lambda i: (0, i)),
        out_shape=jax.ShapeDtypeStruct((B, N), jnp.float32),
        scratch_shapes=[
            pltpu.VMEM((ng * R, 24 * B), jnp.float32),
            pltpu.VMEM((N, 128), jnp.float32),
        ],
        compiler_params=pltpu.CompilerParams(
            dimension_semantics=("arbitrary",),
        ),
    )(e, e, x, wax, wbx, pmat, ssel, bp, lwsel, lb2)

    return y2[:, :, None]
